# baseline (speedup 1.0000x reference)
import functools

import jax
import jax.numpy as jnp
from jax import lax
from jax.experimental import pallas as pl
from jax.experimental.pallas import tpu as pltpu
from jax.experimental.pallas import tpu_sc as plsc


def _make_sc_copy(sl, dim, dtype):
    info = plsc.get_sparse_core_info()
    nw = info.num_cores * info.num_subcores
    rows_per_w = sl // nw
    chunk = 64
    nchunks = rows_per_w // chunk
    mesh = plsc.VectorSubcoreMesh(core_axis_name="c", subcore_axis_name="s")

    @functools.partial(
        pl.kernel,
        mesh=mesh,
        out_type=jax.ShapeDtypeStruct((sl, dim), dtype),
        scratch_types=[pltpu.SemaphoreType.DMA],
    )
    def sc_copy(table_hbm, out_hbm, sem):
        wid = lax.axis_index("s") * info.num_cores + lax.axis_index("c")
        base = wid * rows_per_w

        def copy(i):
            sel = pl.ds(base + i * chunk, chunk)
            return pltpu.make_async_copy(table_hbm.at[sel], out_hbm.at[sel], sem)

        for i in range(nchunks):
            copy(i).start()
        for i in range(nchunks):
            copy(i).wait()

    return sc_copy


def kernel(x, emb_weight):
    sl = x.shape[1]
    dim = emb_weight.shape[1]
    return _make_sc_copy(sl, dim, emb_weight.dtype)(emb_weight)


# speedup vs baseline: 24.3449x; 24.3449x over previous
import functools

import jax
import jax.numpy as jnp
from jax import lax
from jax.experimental import pallas as pl
from jax.experimental.pallas import tpu as pltpu
from jax.experimental.pallas import tpu_sc as plsc

_NBUF = 6
_INFLIGHT = 3


def _make_sc_copy(sl, dim, dtype):
    info = plsc.get_sparse_core_info()
    nw = info.num_cores * info.num_subcores
    rows_per_w = sl // nw
    chunk = 16
    nchunks = rows_per_w // chunk
    mesh = plsc.VectorSubcoreMesh(core_axis_name="c", subcore_axis_name="s")

    @functools.partial(
        pl.kernel,
        mesh=mesh,
        out_type=jax.ShapeDtypeStruct((sl, dim), dtype),
        scratch_types=(
            [pltpu.VMEM((chunk, dim), dtype) for _ in range(_NBUF)]
            + [pltpu.SemaphoreType.DMA for _ in range(2 * _NBUF)]
        ),
    )
    def sc_copy(table_hbm, out_hbm, *scratch):
        bufs = scratch[:_NBUF]
        isems = scratch[_NBUF : 2 * _NBUF]
        osems = scratch[2 * _NBUF :]
        wid = lax.axis_index("s") * info.num_cores + lax.axis_index("c")
        base = wid * rows_per_w

        def in_copy(i):
            b = i % _NBUF
            return pltpu.make_async_copy(
                table_hbm.at[pl.ds(base + i * chunk, chunk)], bufs[b], isems[b]
            )

        def out_copy(i):
            b = i % _NBUF
            return pltpu.make_async_copy(
                bufs[b], out_hbm.at[pl.ds(base + i * chunk, chunk)], osems[b]
            )

        for i in range(_INFLIGHT):
            in_copy(i).start()
        for i in range(nchunks):
            in_copy(i).wait()
            out_copy(i).start()
            j = i - _INFLIGHT
            if j >= 0:
                out_copy(j).wait()
            nxt = i + _INFLIGHT
            if nxt < nchunks:
                in_copy(nxt).start()
        for i in range(max(0, nchunks - _INFLIGHT), nchunks):
            out_copy(i).wait()

    return sc_copy


def kernel(x, emb_weight):
    sl = x.shape[1]
    dim = emb_weight.shape[1]
    return _make_sc_copy(sl, dim, emb_weight.dtype)(emb_weight)


# speedup vs baseline: 24.4959x; 1.0062x over previous
import functools

import jax
from jax import lax
from jax.experimental import pallas as pl
from jax.experimental.pallas import tpu as pltpu
from jax.experimental.pallas import tpu_sc as plsc

_NBUF = 6
_INFLIGHT = 3


def _make_sc_copy(sl, dim, dtype):
    info = plsc.get_sparse_core_info()
    nw = info.num_cores * info.num_subcores
    rows_per_w = sl // nw
    chunk = 16
    nchunks = rows_per_w // chunk
    mesh = plsc.VectorSubcoreMesh(core_axis_name="c", subcore_axis_name="s")

    @functools.partial(
        pl.kernel,
        mesh=mesh,
        out_type=jax.ShapeDtypeStruct((sl, dim), dtype),
        scratch_types=(
            [pltpu.VMEM((chunk, dim), dtype) for _ in range(_NBUF)]
            + [pltpu.SemaphoreType.DMA for _ in range(2 * _NBUF)]
        ),
    )
    def sc_copy(table_hbm, out_hbm, *scratch):
        bufs = scratch[:_NBUF]
        isems = scratch[_NBUF : 2 * _NBUF]
        osems = scratch[2 * _NBUF :]
        wid = lax.axis_index("s") * info.num_cores + lax.axis_index("c")
        base = wid * rows_per_w

        def in_copy(i):
            b = i % _NBUF
            return pltpu.make_async_copy(
                table_hbm.at[pl.ds(base + i * chunk, chunk)], bufs[b], isems[b]
            )

        def out_copy(i):
            b = i % _NBUF
            return pltpu.make_async_copy(
                bufs[b], out_hbm.at[pl.ds(base + i * chunk, chunk)], osems[b]
            )

        for i in range(_INFLIGHT):
            in_copy(i).start()
        for i in range(nchunks):
            in_copy(i).wait()
            out_copy(i).start()
            j = i - _INFLIGHT
            if j >= 0:
                out_copy(j).wait()
            nxt = i + _INFLIGHT
            if nxt < nchunks:
                in_copy(nxt).start()
        for i in range(max(0, nchunks - _INFLIGHT), nchunks):
            out_copy(i).wait()

    return sc_copy


def kernel(x, emb_weight):
    sl = x.shape[1]
    dim = emb_weight.shape[1]
    return _make_sc_copy(sl, dim, emb_weight.dtype)(emb_weight)
